# baseline (speedup 1.0000x reference)
import functools

import jax
import jax.numpy as jnp
from jax import lax
from jax.experimental import pallas as pl
from jax.experimental.pallas import tpu as pltpu
from jax.experimental.pallas import tpu_sc as plsc

N = 10000
D = 128
E = 320000
NC, NS = 2, 16
NW = NC * NS
K = 128
NB = 80
KP = 32
NP = 320
NP0, NP1 = 160, 160
NRING = 8
PS = 448
NA = N + PS
ZR = 208
RPS = 624


@functools.cache
def _sc_mesh():
    return plsc.VectorSubcoreMesh(
        core_axis_name="c", subcore_axis_name="s", num_cores=NC, num_subcores=NS)


def _zero_rows(ref, nrows, width):
    z = jnp.zeros((16,), jnp.float32)

    def body(i, carry):
        for k in range(width // 16):
            ref[i, pl.ds(16 * k, 16)] = z
        return carry

    lax.fori_loop(0, nrows, body, 0)


def _copy_out(c, s, acc, out_hbm):
    r0 = s * RPS
    pltpu.sync_copy(acc.at[pl.ds(r0, RPS)], out_hbm.at[c, pl.ds(r0, RPS)])

    @pl.when(s == NS - 1)
    def _():
        pltpu.sync_copy(acc.at[pl.ds(NS * RPS, N - NS * RPS)],
                        out_hbm.at[c, pl.ds(NS * RPS, N - NS * RPS)])


def _deg_body(dst_hbm, out_hbm, didx, onesb, zed, accd):
    c = lax.axis_index("c")
    s = lax.axis_index("s")
    g = c * NS + s

    one = jnp.ones((16,), jnp.float32)

    def fill(i, carry):
        onesb[i, :] = one
        return carry

    lax.fori_loop(0, KP, fill, 0)
    _zero_rows(zed, ZR, 16)
    r0 = s * RPS
    for t in range(3):
        pltpu.sync_copy(zed, accd.at[pl.ds(r0 + t * ZR, ZR)])

    @pl.when(s == NS - 1)
    def _():
        pltpu.sync_copy(zed.at[pl.ds(0, N - NS * RPS)],
                        accd.at[pl.ds(NS * RPS, N - NS * RPS)])

    plsc.subcore_barrier()

    pltpu.sync_copy(dst_hbm.at[pl.ds(g * NP, NP)], didx)

    def body(i, carry):
        pltpu.sync_copy(onesb, accd.at[didx.at[i]], add=True)
        return carry

    lax.fori_loop(0, NP, body, 0)

    plsc.subcore_barrier()
    _copy_out(c, s, accd, out_hbm)


@functools.cache
def _deg_call():
    return pl.kernel(
        _deg_body,
        out_type=jax.ShapeDtypeStruct((NC, N, 16), jnp.float32),
        mesh=_sc_mesh(),
        compiler_params=pltpu.CompilerParams(use_tc_tiling_on_sc=False),
        scratch_types=[
            pltpu.VMEM((NP, KP), jnp.int32),
            pltpu.VMEM((KP, 16), jnp.float32),
            pltpu.VMEM((ZR, 16), jnp.float32),
            pltpu.VMEM_SHARED((NA, 16), jnp.float32),
        ],
    )


def _prop_body(y_hbm, src_hbm, dst_hbm, out_hbm,
               sidx, didx, buf0, buf1, buf2, buf3, buf4, buf5, buf6, buf7, acc,
               gs0, gs1, gs2, gs3, gs4, gs5, gs6, gs7,
               ss0, ss1, ss2, ss3, ss4, ss5, ss6, ss7):
    c = lax.axis_index("c")
    s = lax.axis_index("s")
    base = c * (NS * NP) + s * NP

    bufs = (buf0, buf1, buf2, buf3, buf4, buf5, buf6, buf7)
    gsems = (gs0, gs1, gs2, gs3, gs4, gs5, gs6, gs7)
    ssems = (ss0, ss1, ss2, ss3, ss4, ss5, ss6, ss7)

    for b in range(NRING):
        _zero_rows(bufs[b], KP, D)
    r0 = s * RPS
    nz = RPS // KP
    rem = RPS - nz * KP
    for t in range(nz):
        b = t % NRING
        pltpu.async_copy(bufs[b], acc.at[pl.ds(r0 + t * KP, KP)], gsems[b])
    if rem:
        br = nz % NRING
        pltpu.async_copy(bufs[br].at[pl.ds(0, rem)],
                         acc.at[pl.ds(r0 + nz * KP, rem)], gsems[br])
    for t in range(nz):
        b = t % NRING
        pltpu.make_async_copy(bufs[b], acc.at[pl.ds(0, KP)], gsems[b]).wait()
    if rem:
        br = nz % NRING
        pltpu.make_async_copy(bufs[br].at[pl.ds(0, rem)],
                              acc.at[pl.ds(0, rem)], gsems[br]).wait()

    @pl.when(s == NS - 1)
    def _():
        pltpu.sync_copy(buf0.at[pl.ds(0, N - NS * RPS)],
                        acc.at[pl.ds(NS * RPS, N - NS * RPS)])

    plsc.subcore_barrier()

    def g_start(i, b):
        pltpu.async_copy(y_hbm.at[sidx.at[i]], bufs[b], gsems[b])

    def g_wait(b):
        pltpu.make_async_copy(y_hbm.at[sidx.at[0]], bufs[b], gsems[b]).wait()

    def s_start(i, b):
        pltpu.async_copy(bufs[b], acc.at[didx.at[i]], ssems[b], add=True)

    def s_wait(b):
        pltpu.make_async_copy(bufs[b], acc.at[didx.at[0]], ssems[b]).wait()

    off = 0
    for np_ in (NP0, NP1):
        pltpu.sync_copy(src_hbm.at[pl.ds(base + off, np_)],
                        sidx.at[pl.ds(0, np_)])
        pltpu.sync_copy(dst_hbm.at[pl.ds(base + off, np_)],
                        didx.at[pl.ds(0, np_)])

        for b in range(NRING):
            g_start(b, b)

        def body(j, carry):
            a = NRING * j
            for b in range(NRING):
                g_wait(b)
                s_start(a + b, b)
            for b in range(NRING):
                s_wait(b)
                g_start(a + NRING + b, b)
            return carry

        lax.fori_loop(0, np_ // NRING - 1, body, 0)

        a = np_ - NRING
        for b in range(NRING):
            g_wait(b)
            s_start(a + b, b)
        for b in range(NRING):
            s_wait(b)
        off += np_

    plsc.subcore_barrier()
    _copy_out(c, s, acc, out_hbm)


@functools.cache
def _prop_call():
    return pl.kernel(
        _prop_body,
        out_type=jax.ShapeDtypeStruct((NC, N, D), jnp.float32),
        mesh=_sc_mesh(),
        compiler_params=pltpu.CompilerParams(use_tc_tiling_on_sc=False),
        scratch_types=[
            pltpu.VMEM((NP0, KP), jnp.int32),
            pltpu.VMEM((NP0, KP), jnp.int32),
        ] + [pltpu.VMEM((KP, D), jnp.float32)] * NRING + [
            pltpu.VMEM_SHARED((NA, D), jnp.float32),
        ] + [pltpu.SemaphoreType.DMA] * (2 * NRING),
    )


_BR = 1000
_GRID = N // _BR


def _tc1_body(degp_ref, x_ref, w1_ref, y1_ref, dinv_ref):
    deg16 = degp_ref[0] + degp_ref[1] + 1.0
    dinv16 = lax.rsqrt(deg16)
    dinv = dinv16[:, 0:1]
    dinv_ref[...] = dinv
    y1_ref[...] = jnp.dot(x_ref[...], w1_ref[...],
                          preferred_element_type=jnp.float32) * dinv


_tc1_call = pl.pallas_call(
    _tc1_body,
    grid=(_GRID,),
    in_specs=[
        pl.BlockSpec((NC, _BR, 16), lambda i: (0, i, 0)),
        pl.BlockSpec((_BR, D), lambda i: (i, 0)),
        pl.BlockSpec((D, D), lambda i: (0, 0)),
    ],
    out_specs=[
        pl.BlockSpec((_BR, D), lambda i: (i, 0)),
        pl.BlockSpec((_BR, 1), lambda i: (i, 0)),
    ],
    out_shape=[
        jax.ShapeDtypeStruct((N, D), jnp.float32),
        jax.ShapeDtypeStruct((N, 1), jnp.float32),
    ],
)


def _tc2_body(s_ref, y1_ref, dinv_ref, b1_ref, y2_ref):
    dinv = dinv_ref[...]
    p = (s_ref[0] + s_ref[1] + y1_ref[...]) * dinv + b1_ref[...]
    y2_ref[...] = jnp.maximum(p, 0.0) * dinv


_tc2_call = pl.pallas_call(
    _tc2_body,
    grid=(_GRID,),
    in_specs=[
        pl.BlockSpec((NC, _BR, D), lambda i: (0, i, 0)),
        pl.BlockSpec((_BR, D), lambda i: (i, 0)),
        pl.BlockSpec((_BR, 1), lambda i: (i, 0)),
        pl.BlockSpec((1, D), lambda i: (0, 0)),
    ],
    out_specs=pl.BlockSpec((_BR, D), lambda i: (i, 0)),
    out_shape=jax.ShapeDtypeStruct((N, D), jnp.float32),
)


def _tc3_body(s_ref, y2_ref, dinv_ref, wmu_ref, wlv_ref, bmu_ref, blv_ref,
              zmu_ref, zlv_ref):
    p2 = (s_ref[0] + s_ref[1] + y2_ref[...]) * dinv_ref[...]
    zmu_ref[...] = jnp.dot(p2, wmu_ref[...],
                           preferred_element_type=jnp.float32) + bmu_ref[...]
    zlv_ref[...] = jnp.dot(p2, wlv_ref[...],
                           preferred_element_type=jnp.float32) + blv_ref[...]


_tc3_call = pl.pallas_call(
    _tc3_body,
    grid=(_GRID,),
    in_specs=[
        pl.BlockSpec((NC, _BR, D), lambda i: (0, i, 0)),
        pl.BlockSpec((_BR, D), lambda i: (i, 0)),
        pl.BlockSpec((_BR, 1), lambda i: (i, 0)),
        pl.BlockSpec((D, D), lambda i: (0, 0)),
        pl.BlockSpec((D, D), lambda i: (0, 0)),
        pl.BlockSpec((1, D), lambda i: (0, 0)),
        pl.BlockSpec((1, D), lambda i: (0, 0)),
    ],
    out_specs=[
        pl.BlockSpec((_BR, D), lambda i: (i, 0)),
        pl.BlockSpec((_BR, D), lambda i: (i, 0)),
    ],
    out_shape=[
        jax.ShapeDtypeStruct((N, D), jnp.float32),
        jax.ShapeDtypeStruct((N, D), jnp.float32),
    ],
)


def kernel(x, edge_index, W1, b1, W_mu, b_mu, W_lv, b_lv):
    ei = edge_index.astype(jnp.int32)
    pad = NW * NP * KP - E
    r = jnp.arange(pad, dtype=jnp.int32)
    src2 = jnp.concatenate([ei[0], r % N]).reshape(NW * NP, KP)
    dst2 = jnp.concatenate([ei[1], N + (r % PS)]).reshape(NW * NP, KP)

    degp = _deg_call()(dst2)
    y1, dinv = _tc1_call(degp, x, W1)
    s1 = _prop_call()(y1, src2, dst2)
    y2 = _tc2_call(s1, y1, dinv, b1.reshape(1, D))
    s2 = _prop_call()(y2, src2, dst2)
    z_mu, z_lv = _tc3_call(s2, y2, dinv, W_mu, W_lv,
                           b_mu.reshape(1, D), b_lv.reshape(1, D))
    return (z_mu, z_lv)


# speedup vs baseline: 1.0361x; 1.0361x over previous
import functools

import jax
import jax.numpy as jnp
from jax import lax
from jax.experimental import pallas as pl
from jax.experimental.pallas import tpu as pltpu
from jax.experimental.pallas import tpu_sc as plsc

N = 10000
D = 128
E = 320000
NC, NS = 2, 16
NW = NC * NS
K = 128
NB = 80
KP = 48
NP = 210
NP0, NP1 = 108, 102
NRING = 6
PS = 448
NA = N + PS
ZR = 208
RPS = 624


@functools.cache
def _sc_mesh():
    return plsc.VectorSubcoreMesh(
        core_axis_name="c", subcore_axis_name="s", num_cores=NC, num_subcores=NS)


def _zero_rows(ref, nrows, width):
    z = jnp.zeros((16,), jnp.float32)

    def body(i, carry):
        for k in range(width // 16):
            ref[i, pl.ds(16 * k, 16)] = z
        return carry

    lax.fori_loop(0, nrows, body, 0)


def _copy_out(c, s, acc, out_hbm):
    r0 = s * RPS
    pltpu.sync_copy(acc.at[pl.ds(r0, RPS)], out_hbm.at[c, pl.ds(r0, RPS)])

    @pl.when(s == NS - 1)
    def _():
        pltpu.sync_copy(acc.at[pl.ds(NS * RPS, N - NS * RPS)],
                        out_hbm.at[c, pl.ds(NS * RPS, N - NS * RPS)])


def _deg_body(dst_hbm, out_hbm, didx, onesb, zed, accd):
    c = lax.axis_index("c")
    s = lax.axis_index("s")
    g = c * NS + s

    one = jnp.ones((16,), jnp.float32)

    def fill(i, carry):
        onesb[i, :] = one
        return carry

    lax.fori_loop(0, KP, fill, 0)
    _zero_rows(zed, ZR, 16)
    r0 = s * RPS
    for t in range(3):
        pltpu.sync_copy(zed, accd.at[pl.ds(r0 + t * ZR, ZR)])

    @pl.when(s == NS - 1)
    def _():
        pltpu.sync_copy(zed.at[pl.ds(0, N - NS * RPS)],
                        accd.at[pl.ds(NS * RPS, N - NS * RPS)])

    plsc.subcore_barrier()

    pltpu.sync_copy(dst_hbm.at[pl.ds(g * NP, NP)], didx)

    def body(i, carry):
        pltpu.sync_copy(onesb, accd.at[didx.at[i]], add=True)
        return carry

    lax.fori_loop(0, NP, body, 0)

    plsc.subcore_barrier()
    _copy_out(c, s, accd, out_hbm)


@functools.cache
def _deg_call():
    return pl.kernel(
        _deg_body,
        out_type=jax.ShapeDtypeStruct((NC, N, 16), jnp.float32),
        mesh=_sc_mesh(),
        compiler_params=pltpu.CompilerParams(use_tc_tiling_on_sc=False),
        scratch_types=[
            pltpu.VMEM((NP, KP), jnp.int32),
            pltpu.VMEM((KP, 16), jnp.float32),
            pltpu.VMEM((ZR, 16), jnp.float32),
            pltpu.VMEM_SHARED((NA, 16), jnp.float32),
        ],
    )


def _prop_body(y_hbm, src_hbm, dst_hbm, out_hbm,
               sidx, didx, buf0, buf1, buf2, buf3, buf4, buf5, acc,
               gs0, gs1, gs2, gs3, gs4, gs5, ss0, ss1, ss2, ss3, ss4, ss5):
    c = lax.axis_index("c")
    s = lax.axis_index("s")
    base = c * (NS * NP) + s * NP

    bufs = (buf0, buf1, buf2, buf3, buf4, buf5)
    gsems = (gs0, gs1, gs2, gs3, gs4, gs5)
    ssems = (ss0, ss1, ss2, ss3, ss4, ss5)

    for b in range(NRING):
        _zero_rows(bufs[b], KP, D)
    r0 = s * RPS
    nz = RPS // KP
    rem = RPS - nz * KP
    for t in range(nz):
        b = t % NRING
        pltpu.async_copy(bufs[b], acc.at[pl.ds(r0 + t * KP, KP)], gsems[b])
    if rem:
        br = nz % NRING
        pltpu.async_copy(bufs[br].at[pl.ds(0, rem)],
                         acc.at[pl.ds(r0 + nz * KP, rem)], gsems[br])
    for t in range(nz):
        b = t % NRING
        pltpu.make_async_copy(bufs[b], acc.at[pl.ds(0, KP)], gsems[b]).wait()
    if rem:
        br = nz % NRING
        pltpu.make_async_copy(bufs[br].at[pl.ds(0, rem)],
                              acc.at[pl.ds(0, rem)], gsems[br]).wait()

    @pl.when(s == NS - 1)
    def _():
        pltpu.sync_copy(buf0.at[pl.ds(0, N - NS * RPS)],
                        acc.at[pl.ds(NS * RPS, N - NS * RPS)])

    plsc.subcore_barrier()

    def g_start(i, b):
        pltpu.async_copy(y_hbm.at[sidx.at[i]], bufs[b], gsems[b])

    def g_wait(b):
        pltpu.make_async_copy(y_hbm.at[sidx.at[0]], bufs[b], gsems[b]).wait()

    def s_start(i, b):
        pltpu.async_copy(bufs[b], acc.at[didx.at[i]], ssems[b], add=True)

    def s_wait(b):
        pltpu.make_async_copy(bufs[b], acc.at[didx.at[0]], ssems[b]).wait()

    off = 0
    for np_ in (NP0, NP1):
        pltpu.sync_copy(src_hbm.at[pl.ds(base + off, np_)],
                        sidx.at[pl.ds(0, np_)])
        pltpu.sync_copy(dst_hbm.at[pl.ds(base + off, np_)],
                        didx.at[pl.ds(0, np_)])

        for b in range(NRING):
            g_start(b, b)

        def body(j, carry):
            a = NRING * j
            for b in range(NRING):
                g_wait(b)
                s_start(a + b, b)
            for b in range(NRING):
                s_wait(b)
                g_start(a + NRING + b, b)
            return carry

        lax.fori_loop(0, np_ // NRING - 1, body, 0)

        a = np_ - NRING
        for b in range(NRING):
            g_wait(b)
            s_start(a + b, b)
        for b in range(NRING):
            s_wait(b)
        off += np_

    plsc.subcore_barrier()
    _copy_out(c, s, acc, out_hbm)


@functools.cache
def _prop_call():
    return pl.kernel(
        _prop_body,
        out_type=jax.ShapeDtypeStruct((NC, N, D), jnp.float32),
        mesh=_sc_mesh(),
        compiler_params=pltpu.CompilerParams(use_tc_tiling_on_sc=False),
        scratch_types=[
            pltpu.VMEM((NP0, KP), jnp.int32),
            pltpu.VMEM((NP0, KP), jnp.int32),
        ] + [pltpu.VMEM((KP, D), jnp.float32)] * NRING + [
            pltpu.VMEM_SHARED((NA, D), jnp.float32),
        ] + [pltpu.SemaphoreType.DMA] * (2 * NRING),
    )


_BR = 1000
_GRID = N // _BR


def _tc1_body(degp_ref, x_ref, w1_ref, y1_ref, dinv_ref):
    deg16 = degp_ref[0] + degp_ref[1] + 1.0
    dinv16 = lax.rsqrt(deg16)
    dinv = dinv16[:, 0:1]
    dinv_ref[...] = dinv
    y1_ref[...] = jnp.dot(x_ref[...], w1_ref[...],
                          preferred_element_type=jnp.float32) * dinv


_tc1_call = pl.pallas_call(
    _tc1_body,
    grid=(_GRID,),
    in_specs=[
        pl.BlockSpec((NC, _BR, 16), lambda i: (0, i, 0)),
        pl.BlockSpec((_BR, D), lambda i: (i, 0)),
        pl.BlockSpec((D, D), lambda i: (0, 0)),
    ],
    out_specs=[
        pl.BlockSpec((_BR, D), lambda i: (i, 0)),
        pl.BlockSpec((_BR, 1), lambda i: (i, 0)),
    ],
    out_shape=[
        jax.ShapeDtypeStruct((N, D), jnp.float32),
        jax.ShapeDtypeStruct((N, 1), jnp.float32),
    ],
)


def _tc2_body(s_ref, y1_ref, dinv_ref, b1_ref, y2_ref):
    dinv = dinv_ref[...]
    p = (s_ref[0] + s_ref[1] + y1_ref[...]) * dinv + b1_ref[...]
    y2_ref[...] = jnp.maximum(p, 0.0) * dinv


_tc2_call = pl.pallas_call(
    _tc2_body,
    grid=(_GRID,),
    in_specs=[
        pl.BlockSpec((NC, _BR, D), lambda i: (0, i, 0)),
        pl.BlockSpec((_BR, D), lambda i: (i, 0)),
        pl.BlockSpec((_BR, 1), lambda i: (i, 0)),
        pl.BlockSpec((1, D), lambda i: (0, 0)),
    ],
    out_specs=pl.BlockSpec((_BR, D), lambda i: (i, 0)),
    out_shape=jax.ShapeDtypeStruct((N, D), jnp.float32),
)


def _tc3_body(s_ref, y2_ref, dinv_ref, wmu_ref, wlv_ref, bmu_ref, blv_ref,
              zmu_ref, zlv_ref):
    p2 = (s_ref[0] + s_ref[1] + y2_ref[...]) * dinv_ref[...]
    zmu_ref[...] = jnp.dot(p2, wmu_ref[...],
                           preferred_element_type=jnp.float32) + bmu_ref[...]
    zlv_ref[...] = jnp.dot(p2, wlv_ref[...],
                           preferred_element_type=jnp.float32) + blv_ref[...]


_tc3_call = pl.pallas_call(
    _tc3_body,
    grid=(_GRID,),
    in_specs=[
        pl.BlockSpec((NC, _BR, D), lambda i: (0, i, 0)),
        pl.BlockSpec((_BR, D), lambda i: (i, 0)),
        pl.BlockSpec((_BR, 1), lambda i: (i, 0)),
        pl.BlockSpec((D, D), lambda i: (0, 0)),
        pl.BlockSpec((D, D), lambda i: (0, 0)),
        pl.BlockSpec((1, D), lambda i: (0, 0)),
        pl.BlockSpec((1, D), lambda i: (0, 0)),
    ],
    out_specs=[
        pl.BlockSpec((_BR, D), lambda i: (i, 0)),
        pl.BlockSpec((_BR, D), lambda i: (i, 0)),
    ],
    out_shape=[
        jax.ShapeDtypeStruct((N, D), jnp.float32),
        jax.ShapeDtypeStruct((N, D), jnp.float32),
    ],
)


def kernel(x, edge_index, W1, b1, W_mu, b_mu, W_lv, b_lv):
    ei = edge_index.astype(jnp.int32)
    pad = NW * NP * KP - E
    r = jnp.arange(pad, dtype=jnp.int32)
    src2 = jnp.concatenate([ei[0], r % N]).reshape(NW * NP, KP)
    dst2 = jnp.concatenate([ei[1], N + (r % PS)]).reshape(NW * NP, KP)

    degp = _deg_call()(dst2)
    y1, dinv = _tc1_call(degp, x, W1)
    s1 = _prop_call()(y1, src2, dst2)
    y2 = _tc2_call(s1, y1, dinv, b1.reshape(1, D))
    s2 = _prop_call()(y2, src2, dst2)
    z_mu, z_lv = _tc3_call(s2, y2, dinv, W_mu, W_lv,
                           b_mu.reshape(1, D), b_lv.reshape(1, D))
    return (z_mu, z_lv)
